# Initial kernel scaffold; baseline (speedup 1.0000x reference)
import functools

import jax
import jax.numpy as jnp
from jax import lax
from jax.experimental import pallas as pl
from jax.experimental.pallas import tpu as pltpu
from jax.experimental.pallas import tpu_sc as plsc

N_NODES = 10000
N_EDGES = 320000
D = 128

NC = 2
NS = 16
NW = NC * NS

CHUNK = 128
CHUNKS_PER_W = 79
EDGES_PER_W = CHUNKS_PER_W * CHUNK
PADDED_EDGES = NW * EDGES_PER_W
ACC_ROWS = 10016
ROWS_PER_TILE = ACC_ROWS // NS
OUT_ROWS_PER_TILE = N_NODES // NS
DEG_W = 16


def _sc_body(src_hbm, dst_hbm, feat_hbm, zfeat_hbm, zdeg_hbm, ones_hbm,
             acc_out, deg_out,
             src_v, dst_v, rows_v, ones_v, acc_sh, deg_sh, sem):
    c = lax.axis_index("c")
    s = lax.axis_index("s")
    wid = s * NC + c

    pltpu.sync_copy(zfeat_hbm, acc_sh.at[pl.ds(s * ROWS_PER_TILE, ROWS_PER_TILE)])
    pltpu.sync_copy(zdeg_hbm, deg_sh.at[pl.ds(s * ROWS_PER_TILE, ROWS_PER_TILE)])
    pltpu.sync_copy(ones_hbm, ones_v)
    pltpu.sync_copy(src_hbm.at[wid], src_v)
    pltpu.sync_copy(dst_hbm.at[wid], dst_v)
    plsc.subcore_barrier()

    def chunk_body(j, carry):
        pltpu.async_copy(feat_hbm.at[src_v.at[j]], rows_v, sem).wait()
        pltpu.sync_copy(rows_v, acc_sh.at[dst_v.at[j]], add=True)
        pltpu.sync_copy(ones_v, deg_sh.at[dst_v.at[j]], add=True)
        return carry

    lax.fori_loop(0, CHUNKS_PER_W, chunk_body, 0)
    plsc.subcore_barrier()

    o0 = s * OUT_ROWS_PER_TILE
    pltpu.sync_copy(acc_sh.at[pl.ds(o0, OUT_ROWS_PER_TILE)],
                    acc_out.at[c, pl.ds(o0, OUT_ROWS_PER_TILE)])
    pltpu.sync_copy(deg_sh.at[pl.ds(o0, OUT_ROWS_PER_TILE)],
                    deg_out.at[c, pl.ds(o0, OUT_ROWS_PER_TILE)])


_sc_aggregate = pl.kernel(
    _sc_body,
    out_type=(
        jax.ShapeDtypeStruct((NC, N_NODES, D), jnp.float32),
        jax.ShapeDtypeStruct((NC, N_NODES, DEG_W), jnp.float32),
    ),
    mesh=plsc.VectorSubcoreMesh(core_axis_name="c", subcore_axis_name="s"),
    scratch_types=[
        pltpu.VMEM((CHUNKS_PER_W, CHUNK), jnp.int32),
        pltpu.VMEM((CHUNKS_PER_W, CHUNK), jnp.int32),
        pltpu.VMEM((CHUNK, D), jnp.float32),
        pltpu.VMEM((CHUNK, DEG_W), jnp.float32),
        pltpu.VMEM_SHARED((ACC_ROWS, D), jnp.float32),
        pltpu.VMEM_SHARED((ACC_ROWS, DEG_W), jnp.float32),
        pltpu.SemaphoreType.DMA,
    ],
)


def _tc_body(feat_ref, s0_ref, s1_ref, d0_ref, d1_ref, wn_ref, ws_ref, b_ref,
             o_ref):
    ssum = s0_ref[...] + s1_ref[...]
    deg = d0_ref[:, :1] + d1_ref[:, :1]
    hn = ssum * (1.0 / jnp.maximum(deg, 1.0))
    x = feat_ref[...]
    dn = (((1,), (1,)), ((), ()))
    o_ref[...] = (
        lax.dot_general(x, ws_ref[...], dn, preferred_element_type=jnp.float32)
        + lax.dot_general(hn, wn_ref[...], dn, preferred_element_type=jnp.float32)
        + b_ref[...]
    )


ROW_BLK = 1000

_tc_combine = pl.pallas_call(
    _tc_body,
    grid=(N_NODES // ROW_BLK,),
    in_specs=[
        pl.BlockSpec((ROW_BLK, D), lambda i: (i, 0)),
        pl.BlockSpec((ROW_BLK, D), lambda i: (i, 0)),
        pl.BlockSpec((ROW_BLK, D), lambda i: (i, 0)),
        pl.BlockSpec((ROW_BLK, DEG_W), lambda i: (i, 0)),
        pl.BlockSpec((ROW_BLK, DEG_W), lambda i: (i, 0)),
        pl.BlockSpec((D, D), lambda i: (0, 0)),
        pl.BlockSpec((D, D), lambda i: (0, 0)),
        pl.BlockSpec((1, D), lambda i: (0, 0)),
    ],
    out_specs=pl.BlockSpec((ROW_BLK, D), lambda i: (i, 0)),
    out_shape=jax.ShapeDtypeStruct((N_NODES, D), jnp.float32),
)


@jax.jit
def kernel(feat, edge_index, W_neigh, W_self, b_self):
    src = edge_index[0]
    dst = edge_index[1]
    pad = PADDED_EDGES - N_EDGES
    src_p = jnp.concatenate([src, jnp.zeros((pad,), jnp.int32)])
    dst_p = jnp.concatenate([dst, jnp.full((pad,), N_NODES, jnp.int32)])
    src_p = src_p.reshape(NW, CHUNKS_PER_W, CHUNK)
    dst_p = dst_p.reshape(NW, CHUNKS_PER_W, CHUNK)

    zfeat = jnp.zeros((ROWS_PER_TILE, D), jnp.float32)
    zdeg = jnp.zeros((ROWS_PER_TILE, DEG_W), jnp.float32)
    ones = jnp.ones((CHUNK, DEG_W), jnp.float32)

    acc, deg = _sc_aggregate(src_p, dst_p, feat, zfeat, zdeg, ones)

    return _tc_combine(feat, acc[0], acc[1], deg[0], deg[1],
                       W_neigh, W_self, b_self.reshape(1, D))


# speedup vs baseline: 6.6796x; 6.6796x over previous
import jax
import jax.numpy as jnp
from jax import lax
from jax.experimental import pallas as pl
from jax.experimental.pallas import tpu as pltpu
from jax.experimental.pallas import tpu_sc as plsc

N_NODES = 10000
N_EDGES = 320000
D = 128
DH = D // 2

NC = 2
NS = 16
NW = NC * NS

CHUNK = 128
CHUNKS_PER_T = 157
EDGES_PER_T = CHUNKS_PER_T * CHUNK
PADDED_EDGES = NS * EDGES_PER_T
ACC_ROWS = 10112
ROWS_PER_TILE = ACC_ROWS // NS
DEG_W = 8


def _sc_body(src_hbm, dst_hbm, featL_hbm, featR_hbm, zdeg_hbm, ones_hbm,
             acc_out, deg_out,
             src_v, dst_v, rows_v, ones_v, acc_sh, deg_sh, sem):
    c = lax.axis_index("c")
    s = lax.axis_index("s")

    zv = jnp.zeros((16,), jnp.float32)

    def zrow(i, carry):
        for l in range(DH // 16):
            rows_v[i, pl.ds(l * 16, 16)] = zv
        return carry

    lax.fori_loop(0, CHUNK, zrow, 0)
    o0 = s * ROWS_PER_TILE
    for k in range(4):
        pltpu.sync_copy(rows_v, acc_sh.at[pl.ds(o0 + k * CHUNK, CHUNK)])
    pltpu.sync_copy(rows_v.at[pl.ds(0, ROWS_PER_TILE - 4 * CHUNK)],
                    acc_sh.at[pl.ds(o0 + 4 * CHUNK, ROWS_PER_TILE - 4 * CHUNK)])
    pltpu.sync_copy(zdeg_hbm, deg_sh.at[pl.ds(o0, ROWS_PER_TILE)])
    pltpu.sync_copy(ones_hbm, ones_v)
    pltpu.sync_copy(src_hbm.at[s], src_v)
    pltpu.sync_copy(dst_hbm.at[s], dst_v)
    plsc.subcore_barrier()

    def chunk_body(j, carry):
        @pl.when(c == 0)
        def _():
            pltpu.async_copy(featL_hbm.at[src_v.at[j]], rows_v, sem).wait()

        @pl.when(c == 1)
        def _():
            pltpu.async_copy(featR_hbm.at[src_v.at[j]], rows_v, sem).wait()

        pltpu.sync_copy(rows_v, acc_sh.at[dst_v.at[j]], add=True)
        return carry

    lax.fori_loop(0, CHUNKS_PER_T, chunk_body, 0)

    def deg_body(jj, carry):
        pltpu.sync_copy(ones_v, deg_sh.at[dst_v.at[2 * jj + c]], add=True)
        return carry

    n_deg = (CHUNKS_PER_T + 1) // 2
    lax.fori_loop(0, n_deg - c, deg_body, 0)
    plsc.subcore_barrier()

    pltpu.sync_copy(acc_sh.at[pl.ds(o0, ROWS_PER_TILE)],
                    acc_out.at[c, pl.ds(o0, ROWS_PER_TILE)])
    pltpu.sync_copy(deg_sh.at[pl.ds(o0, ROWS_PER_TILE)],
                    deg_out.at[c, pl.ds(o0, ROWS_PER_TILE)])


_sc_aggregate = pl.kernel(
    _sc_body,
    out_type=(
        jax.ShapeDtypeStruct((NC, ACC_ROWS, DH), jnp.float32),
        jax.ShapeDtypeStruct((NC, ACC_ROWS, DEG_W), jnp.float32),
    ),
    mesh=plsc.VectorSubcoreMesh(core_axis_name="c", subcore_axis_name="s"),
    compiler_params=pltpu.CompilerParams(use_tc_tiling_on_sc=False),
    scratch_types=[
        pltpu.VMEM((CHUNKS_PER_T, CHUNK), jnp.int32),
        pltpu.VMEM((CHUNKS_PER_T, CHUNK), jnp.int32),
        pltpu.VMEM((CHUNK, DH), jnp.float32),
        pltpu.VMEM((CHUNK, DEG_W), jnp.float32),
        pltpu.VMEM_SHARED((ACC_ROWS, DH), jnp.float32),
        pltpu.VMEM_SHARED((ACC_ROWS, DEG_W), jnp.float32),
        pltpu.SemaphoreType.DMA,
    ],
)


def _tc_body(feat_ref, a0_ref, a1_ref, d0_ref, d1_ref, wnl_ref, wnr_ref,
             ws_ref, b_ref, o_ref):
    deg = d0_ref[:, :1] + d1_ref[:, :1]
    inv = 1.0 / jnp.maximum(deg, 1.0)
    h0 = a0_ref[...] * inv
    h1 = a1_ref[...] * inv
    x = feat_ref[...]
    dn = (((1,), (1,)), ((), ()))
    o_ref[...] = (
        lax.dot_general(x, ws_ref[...], dn, preferred_element_type=jnp.float32)
        + lax.dot_general(h0, wnl_ref[...], dn, preferred_element_type=jnp.float32)
        + lax.dot_general(h1, wnr_ref[...], dn, preferred_element_type=jnp.float32)
        + b_ref[...]
    )


ROW_BLK = 1000

_tc_combine = pl.pallas_call(
    _tc_body,
    grid=(N_NODES // ROW_BLK,),
    in_specs=[
        pl.BlockSpec((ROW_BLK, D), lambda i: (i, 0)),
        pl.BlockSpec((ROW_BLK, DH), lambda i: (i, 0)),
        pl.BlockSpec((ROW_BLK, DH), lambda i: (i, 0)),
        pl.BlockSpec((ROW_BLK, DEG_W), lambda i: (i, 0)),
        pl.BlockSpec((ROW_BLK, DEG_W), lambda i: (i, 0)),
        pl.BlockSpec((D, DH), lambda i: (0, 0)),
        pl.BlockSpec((D, DH), lambda i: (0, 0)),
        pl.BlockSpec((D, D), lambda i: (0, 0)),
        pl.BlockSpec((1, D), lambda i: (0, 0)),
    ],
    out_specs=pl.BlockSpec((ROW_BLK, D), lambda i: (i, 0)),
    out_shape=jax.ShapeDtypeStruct((N_NODES, D), jnp.float32),
)


@jax.jit
def kernel(feat, edge_index, W_neigh, W_self, b_self):
    src = edge_index[0]
    dst = edge_index[1]
    pad = PADDED_EDGES - N_EDGES
    src_p = jnp.concatenate([src, jnp.zeros((pad,), jnp.int32)])
    dst_p = jnp.concatenate([dst, jnp.full((pad,), N_NODES, jnp.int32)])
    src_p = src_p.reshape(NS, CHUNKS_PER_T, CHUNK)
    dst_p = dst_p.reshape(NS, CHUNKS_PER_T, CHUNK)

    featL = feat[:, :DH]
    featR = feat[:, DH:]
    zdeg = jnp.zeros((ROWS_PER_TILE, DEG_W), jnp.float32)
    ones = jnp.ones((CHUNK, DEG_W), jnp.float32)

    acc, deg = _sc_aggregate(src_p, dst_p, featL, featR, zdeg, ones)

    return _tc_combine(feat, acc[0, :N_NODES], acc[1, :N_NODES],
                       deg[0, :N_NODES], deg[1, :N_NODES],
                       W_neigh[:, :DH], W_neigh[:, DH:],
                       W_self, b_self.reshape(1, D))


# speedup vs baseline: 7.9774x; 1.1943x over previous
import jax
import jax.numpy as jnp
from jax import lax
from jax.experimental import pallas as pl
from jax.experimental.pallas import tpu as pltpu
from jax.experimental.pallas import tpu_sc as plsc

N_NODES = 10000
N_EDGES = 320000
D = 128
DH = D // 2

NC = 2
NS = 16
NW = NC * NS

CHUNK = 128
CHUNKS_PER_T = 158
EDGES_PER_T = CHUNKS_PER_T * CHUNK
PADDED_EDGES = NS * EDGES_PER_T
ACC_ROWS = 10112
ROWS_PER_TILE = ACC_ROWS // NS
DEG_W = 8


def _sc_body(src_hbm, dst_hbm, featL_hbm, featR_hbm, zdeg_hbm, ones_hbm,
             acc_out, deg_out,
             src_v, dst_v, rows0_v, rows1_v, ones_v, acc_sh, deg_sh,
             sem0, sem1):
    c = lax.axis_index("c")
    s = lax.axis_index("s")
    bufs = (rows0_v, rows1_v)
    sems = (sem0, sem1)

    zv = jnp.zeros((16,), jnp.float32)

    def zrow(i, carry):
        for l in range(DH // 16):
            rows0_v[i, pl.ds(l * 16, 16)] = zv
        return carry

    lax.fori_loop(0, CHUNK, zrow, 0)
    o0 = s * ROWS_PER_TILE
    for k in range(4):
        pltpu.sync_copy(rows0_v, acc_sh.at[pl.ds(o0 + k * CHUNK, CHUNK)])
    pltpu.sync_copy(rows0_v.at[pl.ds(0, ROWS_PER_TILE - 4 * CHUNK)],
                    acc_sh.at[pl.ds(o0 + 4 * CHUNK, ROWS_PER_TILE - 4 * CHUNK)])
    pltpu.sync_copy(zdeg_hbm, deg_sh.at[pl.ds(o0, ROWS_PER_TILE)])
    pltpu.sync_copy(ones_hbm, ones_v)
    pltpu.sync_copy(src_hbm.at[s], src_v)
    pltpu.sync_copy(dst_hbm.at[s], dst_v)
    plsc.subcore_barrier()

    def gstart(j, b):
        @pl.when(c == 0)
        def _():
            pltpu.async_copy(featL_hbm.at[src_v.at[j]], bufs[b], sems[b])

        @pl.when(c == 1)
        def _():
            pltpu.async_copy(featR_hbm.at[src_v.at[j]], bufs[b], sems[b])

    def gwait(j, b):
        @pl.when(c == 0)
        def _():
            pltpu.make_async_copy(featL_hbm.at[src_v.at[j]], bufs[b],
                                  sems[b]).wait()

        @pl.when(c == 1)
        def _():
            pltpu.make_async_copy(featR_hbm.at[src_v.at[j]], bufs[b],
                                  sems[b]).wait()

    def consume(j, b):
        pltpu.sync_copy(bufs[b], acc_sh.at[dst_v.at[j]], add=True)
        @pl.when(c == b)
        def _():
            pltpu.sync_copy(ones_v, deg_sh.at[dst_v.at[j]], add=True)

    gstart(0, 0)
    gstart(1, 1)

    def chunk_pair(j2, carry):
        for b in range(2):
            j = 2 * j2 + b
            gwait(j, b)
            consume(j, b)
            gstart(j + 2, b)
        return carry

    lax.fori_loop(0, CHUNKS_PER_T // 2 - 1, chunk_pair, 0)
    for b in range(2):
        j = CHUNKS_PER_T - 2 + b
        gwait(j, b)
        consume(j, b)
    plsc.subcore_barrier()

    pltpu.sync_copy(acc_sh.at[pl.ds(o0, ROWS_PER_TILE)],
                    acc_out.at[c, pl.ds(o0, ROWS_PER_TILE)])
    pltpu.sync_copy(deg_sh.at[pl.ds(o0, ROWS_PER_TILE)],
                    deg_out.at[c, pl.ds(o0, ROWS_PER_TILE)])


_sc_aggregate = pl.kernel(
    _sc_body,
    out_type=(
        jax.ShapeDtypeStruct((NC, ACC_ROWS, DH), jnp.float32),
        jax.ShapeDtypeStruct((NC, ACC_ROWS, DEG_W), jnp.float32),
    ),
    mesh=plsc.VectorSubcoreMesh(core_axis_name="c", subcore_axis_name="s"),
    compiler_params=pltpu.CompilerParams(use_tc_tiling_on_sc=False),
    scratch_types=[
        pltpu.VMEM((CHUNKS_PER_T, CHUNK), jnp.int32),
        pltpu.VMEM((CHUNKS_PER_T, CHUNK), jnp.int32),
        pltpu.VMEM((CHUNK, DH), jnp.float32),
        pltpu.VMEM((CHUNK, DH), jnp.float32),
        pltpu.VMEM((CHUNK, DEG_W), jnp.float32),
        pltpu.VMEM_SHARED((ACC_ROWS, DH), jnp.float32),
        pltpu.VMEM_SHARED((ACC_ROWS, DEG_W), jnp.float32),
        pltpu.SemaphoreType.DMA,
        pltpu.SemaphoreType.DMA,
    ],
)


def _tc_body(feat_ref, a0_ref, a1_ref, d0_ref, d1_ref, wnl_ref, wnr_ref,
             ws_ref, b_ref, o_ref):
    deg = d0_ref[:, :1] + d1_ref[:, :1]
    inv = 1.0 / jnp.maximum(deg, 1.0)
    h0 = a0_ref[...] * inv
    h1 = a1_ref[...] * inv
    x = feat_ref[...]
    dn = (((1,), (1,)), ((), ()))
    o_ref[...] = (
        lax.dot_general(x, ws_ref[...], dn, preferred_element_type=jnp.float32)
        + lax.dot_general(h0, wnl_ref[...], dn, preferred_element_type=jnp.float32)
        + lax.dot_general(h1, wnr_ref[...], dn, preferred_element_type=jnp.float32)
        + b_ref[...]
    )


ROW_BLK = 1000

_tc_combine = pl.pallas_call(
    _tc_body,
    grid=(N_NODES // ROW_BLK,),
    in_specs=[
        pl.BlockSpec((ROW_BLK, D), lambda i: (i, 0)),
        pl.BlockSpec((ROW_BLK, DH), lambda i: (i, 0)),
        pl.BlockSpec((ROW_BLK, DH), lambda i: (i, 0)),
        pl.BlockSpec((ROW_BLK, DEG_W), lambda i: (i, 0)),
        pl.BlockSpec((ROW_BLK, DEG_W), lambda i: (i, 0)),
        pl.BlockSpec((D, DH), lambda i: (0, 0)),
        pl.BlockSpec((D, DH), lambda i: (0, 0)),
        pl.BlockSpec((D, D), lambda i: (0, 0)),
        pl.BlockSpec((1, D), lambda i: (0, 0)),
    ],
    out_specs=pl.BlockSpec((ROW_BLK, D), lambda i: (i, 0)),
    out_shape=jax.ShapeDtypeStruct((N_NODES, D), jnp.float32),
)


@jax.jit
def kernel(feat, edge_index, W_neigh, W_self, b_self):
    src = edge_index[0]
    dst = edge_index[1]
    pad = PADDED_EDGES - N_EDGES
    src_p = jnp.concatenate([src, jnp.zeros((pad,), jnp.int32)])
    dst_p = jnp.concatenate([dst, jnp.full((pad,), N_NODES, jnp.int32)])
    src_p = src_p.reshape(NS, CHUNKS_PER_T, CHUNK)
    dst_p = dst_p.reshape(NS, CHUNKS_PER_T, CHUNK)

    featL = feat[:, :DH]
    featR = feat[:, DH:]
    zdeg = jnp.zeros((ROWS_PER_TILE, DEG_W), jnp.float32)
    ones = jnp.ones((CHUNK, DEG_W), jnp.float32)

    acc, deg = _sc_aggregate(src_p, dst_p, featL, featR, zdeg, ones)

    return _tc_combine(feat, acc[0, :N_NODES], acc[1, :N_NODES],
                       deg[0, :N_NODES], deg[1, :N_NODES],
                       W_neigh[:, :DH], W_neigh[:, DH:],
                       W_self, b_self.reshape(1, D))


# speedup vs baseline: 13.6909x; 1.7162x over previous
import jax
import jax.numpy as jnp
from jax import lax
from jax.experimental import pallas as pl
from jax.experimental.pallas import tpu as pltpu
from jax.experimental.pallas import tpu_sc as plsc

N_NODES = 10000
N_EDGES = 320000
D = 128
DH = D // 2

NC = 2
NS = 16

EDGES_PER_T = N_EDGES // NS
CHUNK = 128
CHUNKS_PER_T = 156
TAIL = EDGES_PER_T - CHUNKS_PER_T * CHUNK
ACC_ROWS = 10112
ROWS_PER_TILE = ACC_ROWS // NS
DEG_W = 8
NB = 4


def _sc_body(edge_hbm, table_hbm, zacc_hbm, zdeg_hbm, ones_hbm,
             acc_out, deg_out,
             src_v, dst_v, b0, b1, b2, b3, ones_v, acc_sh, deg_sh,
             g0, g1, g2, g3, s0, s1, s2, s3):
    c = lax.axis_index("c")
    s = lax.axis_index("s")
    bufs = (b0, b1, b2, b3)
    gsem = (g0, g1, g2, g3)
    ssem = (s0, s1, s2, s3)

    o0 = s * ROWS_PER_TILE
    pltpu.sync_copy(zacc_hbm, acc_sh.at[pl.ds(o0, ROWS_PER_TILE)])
    pltpu.sync_copy(zdeg_hbm, deg_sh.at[pl.ds(o0, ROWS_PER_TILE)])
    pltpu.sync_copy(ones_hbm, ones_v)
    e0 = s * EDGES_PER_T
    pltpu.sync_copy(edge_hbm.at[0, pl.ds(e0, EDGES_PER_T)], src_v)
    pltpu.sync_copy(edge_hbm.at[1, pl.ds(e0, EDGES_PER_T)], dst_v)

    def ixform(i, carry):
        v = src_v[pl.ds(i * 16, 16)]
        src_v[pl.ds(i * 16, 16)] = v * 2 + c
        return carry

    lax.fori_loop(0, EDGES_PER_T // 16, ixform, 0)
    plsc.subcore_barrier()

    def gstart(j, b):
        pltpu.async_copy(table_hbm.at[src_v.at[pl.ds(j * CHUNK, CHUNK)]],
                         bufs[b], gsem[b])

    def gwait(j, b):
        pltpu.make_async_copy(table_hbm.at[src_v.at[pl.ds(j * CHUNK, CHUNK)]],
                              bufs[b], gsem[b]).wait()

    def sstart(j, b):
        pltpu.async_copy(bufs[b], acc_sh.at[dst_v.at[pl.ds(j * CHUNK, CHUNK)]],
                         ssem[b], add=True)
        @pl.when(c == (b % 2))
        def _():
            pltpu.sync_copy(ones_v,
                            deg_sh.at[dst_v.at[pl.ds(j * CHUNK, CHUNK)]],
                            add=True)

    def swait(j, b):
        pltpu.make_async_copy(bufs[b],
                              acc_sh.at[dst_v.at[pl.ds(j * CHUNK, CHUNK)]],
                              ssem[b]).wait()

    for j in range(NB):
        gstart(j, j)

    gwait(0, 0)
    sstart(0, 0)
    for b in range(1, NB):
        gwait(b, b)
        sstart(b, b)
        swait(b - 1, b - 1)
        gstart(b + NB - 1, b - 1)

    def ring_shift(j4, carry):
        for b in range(NB):
            j = NB * j4 + b
            prev = (b - 1) % NB
            gwait(j, b)
            sstart(j, b)
            swait(j - 1, prev)
            gstart(j + NB - 1, prev)
        return carry

    lax.fori_loop(1, CHUNKS_PER_T // NB - 1, ring_shift, 0)
    last = CHUNKS_PER_T - NB
    for b in range(NB):
        j = last + b
        prev = (b - 1) % NB
        gwait(j, b)
        sstart(j, b)
        swait(j - 1, prev)
        if b == 0:
            gstart(CHUNKS_PER_T - 1, NB - 1)
    swait(CHUNKS_PER_T - 1, NB - 1)

    t0 = CHUNKS_PER_T * CHUNK
    pltpu.async_copy(table_hbm.at[src_v.at[pl.ds(t0, TAIL)]],
                     b0.at[pl.ds(0, TAIL)], g0).wait()
    pltpu.sync_copy(b0.at[pl.ds(0, TAIL)],
                    acc_sh.at[dst_v.at[pl.ds(t0, TAIL)]], add=True)

    @pl.when(c == 0)
    def _():
        pltpu.sync_copy(ones_v.at[pl.ds(0, TAIL)],
                        deg_sh.at[dst_v.at[pl.ds(t0, TAIL)]], add=True)

    plsc.subcore_barrier()

    pltpu.sync_copy(acc_sh.at[pl.ds(o0, ROWS_PER_TILE)],
                    acc_out.at[c, pl.ds(o0, ROWS_PER_TILE)])
    pltpu.sync_copy(deg_sh.at[pl.ds(o0, ROWS_PER_TILE)],
                    deg_out.at[c, pl.ds(o0, ROWS_PER_TILE)])


_sc_aggregate = pl.kernel(
    _sc_body,
    out_type=(
        jax.ShapeDtypeStruct((NC, ACC_ROWS, DH), jnp.float32),
        jax.ShapeDtypeStruct((NC, ACC_ROWS, DEG_W), jnp.float32),
    ),
    mesh=plsc.VectorSubcoreMesh(core_axis_name="c", subcore_axis_name="s"),
    compiler_params=pltpu.CompilerParams(use_tc_tiling_on_sc=False),
    scratch_types=[
        pltpu.VMEM((EDGES_PER_T,), jnp.int32),
        pltpu.VMEM((EDGES_PER_T,), jnp.int32),
        pltpu.VMEM((CHUNK, DH), jnp.float32),
        pltpu.VMEM((CHUNK, DH), jnp.float32),
        pltpu.VMEM((CHUNK, DH), jnp.float32),
        pltpu.VMEM((CHUNK, DH), jnp.float32),
        pltpu.VMEM((CHUNK, DEG_W), jnp.float32),
        pltpu.VMEM_SHARED((ACC_ROWS, DH), jnp.float32),
        pltpu.VMEM_SHARED((ACC_ROWS, DEG_W), jnp.float32),
        pltpu.SemaphoreType.DMA,
        pltpu.SemaphoreType.DMA,
        pltpu.SemaphoreType.DMA,
        pltpu.SemaphoreType.DMA,
        pltpu.SemaphoreType.DMA,
        pltpu.SemaphoreType.DMA,
        pltpu.SemaphoreType.DMA,
        pltpu.SemaphoreType.DMA,
    ],
)


def _tc_body(feat_ref, acc_ref, deg_ref, wnl_ref, wnr_ref, ws_ref, b_ref,
             o_ref):
    deg = deg_ref[0, :, :1] + deg_ref[1, :, :1]
    inv = 1.0 / jnp.maximum(deg, 1.0)
    h0 = acc_ref[0] * inv
    h1 = acc_ref[1] * inv
    x = feat_ref[...]
    dn = (((1,), (1,)), ((), ()))
    o_ref[...] = (
        lax.dot_general(x, ws_ref[...], dn, preferred_element_type=jnp.float32)
        + lax.dot_general(h0, wnl_ref[...], dn, preferred_element_type=jnp.float32)
        + lax.dot_general(h1, wnr_ref[...], dn, preferred_element_type=jnp.float32)
        + b_ref[...]
    )


ROW_BLK = 1000

_tc_combine = pl.pallas_call(
    _tc_body,
    grid=(N_NODES // ROW_BLK,),
    in_specs=[
        pl.BlockSpec((ROW_BLK, D), lambda i: (i, 0)),
        pl.BlockSpec((NC, ROW_BLK, DH), lambda i: (0, i, 0)),
        pl.BlockSpec((NC, ROW_BLK, DEG_W), lambda i: (0, i, 0)),
        pl.BlockSpec((D, DH), lambda i: (0, 0)),
        pl.BlockSpec((D, DH), lambda i: (0, 0)),
        pl.BlockSpec((D, D), lambda i: (0, 0)),
        pl.BlockSpec((1, D), lambda i: (0, 0)),
    ],
    out_specs=pl.BlockSpec((ROW_BLK, D), lambda i: (i, 0)),
    out_shape=jax.ShapeDtypeStruct((N_NODES, D), jnp.float32),
)


@jax.jit
def kernel(feat, edge_index, W_neigh, W_self, b_self):
    table = feat.reshape(N_NODES * 2, DH)
    zacc = jnp.zeros((ROWS_PER_TILE, DH), jnp.float32)
    zdeg = jnp.zeros((ROWS_PER_TILE, DEG_W), jnp.float32)
    ones = jnp.ones((CHUNK, DEG_W), jnp.float32)

    acc, deg = _sc_aggregate(edge_index, table, zacc, zdeg, ones)

    return _tc_combine(feat, acc, deg,
                       W_neigh[:, :DH], W_neigh[:, DH:],
                       W_self, b_self.reshape(1, D))


# speedup vs baseline: 14.1740x; 1.0353x over previous
import jax
import jax.numpy as jnp
from jax import lax
from jax.experimental import pallas as pl
from jax.experimental.pallas import tpu as pltpu
from jax.experimental.pallas import tpu_sc as plsc

N_NODES = 10000
N_EDGES = 320000
D = 128
DH = D // 2

NC = 2
NS = 16

EDGES_PER_T = N_EDGES // NS
CHUNK = 128
CHUNKS_PER_T = 156
TAIL = EDGES_PER_T - CHUNKS_PER_T * CHUNK
ACC_ROWS = 10112
ROWS_PER_TILE = ACC_ROWS // NS
DEG_W = 8
NB = 4


def _sc_body(edge_hbm, table_hbm, zacc_hbm, zdeg_hbm, ones_hbm,
             acc_out, deg_out,
             src_v, dst_v, b0, b1, b2, b3, ones_v, acc_sh, deg_sh,
             g0, g1, g2, g3, s0, s1, s2, s3, d0, d1, d2, d3):
    c = lax.axis_index("c")
    s = lax.axis_index("s")
    bufs = (b0, b1, b2, b3)
    gsem = (g0, g1, g2, g3)
    ssem = (s0, s1, s2, s3)
    dsem = (d0, d1, d2, d3)

    o0 = s * ROWS_PER_TILE
    pltpu.sync_copy(zacc_hbm, acc_sh.at[pl.ds(o0, ROWS_PER_TILE)])
    pltpu.sync_copy(zdeg_hbm, deg_sh.at[pl.ds(o0, ROWS_PER_TILE)])
    pltpu.sync_copy(ones_hbm, ones_v)
    e0 = s * EDGES_PER_T
    pltpu.sync_copy(edge_hbm.at[0, pl.ds(e0, EDGES_PER_T)], src_v)
    pltpu.sync_copy(edge_hbm.at[1, pl.ds(e0, EDGES_PER_T)], dst_v)

    def ixform(i, carry):
        v = src_v[pl.ds(i * 16, 16)]
        src_v[pl.ds(i * 16, 16)] = v * 2 + c
        return carry

    lax.fori_loop(0, EDGES_PER_T // 16, ixform, 0)
    plsc.subcore_barrier()

    def gstart(j, b):
        pltpu.async_copy(table_hbm.at[src_v.at[pl.ds(j * CHUNK, CHUNK)]],
                         bufs[b], gsem[b])

    def gwait(j, b):
        pltpu.make_async_copy(table_hbm.at[src_v.at[pl.ds(j * CHUNK, CHUNK)]],
                              bufs[b], gsem[b]).wait()

    def sstart(j, b, dwait_j=None):
        pltpu.async_copy(bufs[b], acc_sh.at[dst_v.at[pl.ds(j * CHUNK, CHUNK)]],
                         ssem[b], add=True)
        @pl.when(c == (b % 2))
        def _():
            if dwait_j is not None:
                pltpu.make_async_copy(
                    ones_v, deg_sh.at[dst_v.at[pl.ds(dwait_j * CHUNK, CHUNK)]],
                    dsem[b]).wait()
            pltpu.async_copy(ones_v,
                             deg_sh.at[dst_v.at[pl.ds(j * CHUNK, CHUNK)]],
                             dsem[b], add=True)

    def swait(j, b):
        pltpu.make_async_copy(bufs[b],
                              acc_sh.at[dst_v.at[pl.ds(j * CHUNK, CHUNK)]],
                              ssem[b]).wait()

    for j in range(NB):
        gstart(j, j)

    gwait(0, 0)
    sstart(0, 0)
    for b in range(1, NB):
        gwait(b, b)
        sstart(b, b)
        swait(b - 1, b - 1)
        gstart(b + NB - 1, b - 1)

    def ring_shift(j4, carry):
        for b in range(NB):
            j = NB * j4 + b
            prev = (b - 1) % NB
            gwait(j, b)
            sstart(j, b, dwait_j=j - NB)
            swait(j - 1, prev)
            gstart(j + NB - 1, prev)
        return carry

    lax.fori_loop(1, CHUNKS_PER_T // NB - 1, ring_shift, 0)
    last = CHUNKS_PER_T - NB
    for b in range(NB):
        j = last + b
        prev = (b - 1) % NB
        gwait(j, b)
        sstart(j, b, dwait_j=j - NB)
        swait(j - 1, prev)
        if b == 0:
            gstart(CHUNKS_PER_T - 1, NB - 1)
    swait(CHUNKS_PER_T - 1, NB - 1)
    for b in range(NB):
        @pl.when(c == (b % 2))
        def _(b=b):
            pltpu.make_async_copy(
                ones_v, deg_sh.at[dst_v.at[pl.ds((last + b) * CHUNK, CHUNK)]],
                dsem[b]).wait()

    t0 = CHUNKS_PER_T * CHUNK
    pltpu.async_copy(table_hbm.at[src_v.at[pl.ds(t0, TAIL)]],
                     b0.at[pl.ds(0, TAIL)], g0).wait()
    pltpu.sync_copy(b0.at[pl.ds(0, TAIL)],
                    acc_sh.at[dst_v.at[pl.ds(t0, TAIL)]], add=True)

    @pl.when(c == 0)
    def _():
        pltpu.sync_copy(ones_v.at[pl.ds(0, TAIL)],
                        deg_sh.at[dst_v.at[pl.ds(t0, TAIL)]], add=True)

    plsc.subcore_barrier()

    pltpu.sync_copy(acc_sh.at[pl.ds(o0, ROWS_PER_TILE)],
                    acc_out.at[c, pl.ds(o0, ROWS_PER_TILE)])
    pltpu.sync_copy(deg_sh.at[pl.ds(o0, ROWS_PER_TILE)],
                    deg_out.at[c, pl.ds(o0, ROWS_PER_TILE)])


_sc_aggregate = pl.kernel(
    _sc_body,
    out_type=(
        jax.ShapeDtypeStruct((NC, ACC_ROWS, DH), jnp.float32),
        jax.ShapeDtypeStruct((NC, ACC_ROWS, DEG_W), jnp.float32),
    ),
    mesh=plsc.VectorSubcoreMesh(core_axis_name="c", subcore_axis_name="s"),
    compiler_params=pltpu.CompilerParams(use_tc_tiling_on_sc=False),
    scratch_types=[
        pltpu.VMEM((EDGES_PER_T,), jnp.int32),
        pltpu.VMEM((EDGES_PER_T,), jnp.int32),
        pltpu.VMEM((CHUNK, DH), jnp.float32),
        pltpu.VMEM((CHUNK, DH), jnp.float32),
        pltpu.VMEM((CHUNK, DH), jnp.float32),
        pltpu.VMEM((CHUNK, DH), jnp.float32),
        pltpu.VMEM((CHUNK, DEG_W), jnp.float32),
        pltpu.VMEM_SHARED((ACC_ROWS, DH), jnp.float32),
        pltpu.VMEM_SHARED((ACC_ROWS, DEG_W), jnp.float32),
        pltpu.SemaphoreType.DMA,
        pltpu.SemaphoreType.DMA,
        pltpu.SemaphoreType.DMA,
        pltpu.SemaphoreType.DMA,
        pltpu.SemaphoreType.DMA,
        pltpu.SemaphoreType.DMA,
        pltpu.SemaphoreType.DMA,
        pltpu.SemaphoreType.DMA,
        pltpu.SemaphoreType.DMA,
        pltpu.SemaphoreType.DMA,
        pltpu.SemaphoreType.DMA,
        pltpu.SemaphoreType.DMA,
    ],
)


def _tc_body(feat_ref, acc_ref, deg_ref, wnl_ref, wnr_ref, ws_ref, b_ref,
             o_ref):
    deg = deg_ref[0, :, :1] + deg_ref[1, :, :1]
    inv = 1.0 / jnp.maximum(deg, 1.0)
    h0 = acc_ref[0] * inv
    h1 = acc_ref[1] * inv
    x = feat_ref[...]
    dn = (((1,), (1,)), ((), ()))
    o_ref[...] = (
        lax.dot_general(x, ws_ref[...], dn, preferred_element_type=jnp.float32)
        + lax.dot_general(h0, wnl_ref[...], dn, preferred_element_type=jnp.float32)
        + lax.dot_general(h1, wnr_ref[...], dn, preferred_element_type=jnp.float32)
        + b_ref[...]
    )


ROW_BLK = 2000

_tc_combine = pl.pallas_call(
    _tc_body,
    grid=(N_NODES // ROW_BLK,),
    in_specs=[
        pl.BlockSpec((ROW_BLK, D), lambda i: (i, 0)),
        pl.BlockSpec((NC, ROW_BLK, DH), lambda i: (0, i, 0)),
        pl.BlockSpec((NC, ROW_BLK, DEG_W), lambda i: (0, i, 0)),
        pl.BlockSpec((D, DH), lambda i: (0, 0)),
        pl.BlockSpec((D, DH), lambda i: (0, 0)),
        pl.BlockSpec((D, D), lambda i: (0, 0)),
        pl.BlockSpec((1, D), lambda i: (0, 0)),
    ],
    out_specs=pl.BlockSpec((ROW_BLK, D), lambda i: (i, 0)),
    out_shape=jax.ShapeDtypeStruct((N_NODES, D), jnp.float32),
)


@jax.jit
def kernel(feat, edge_index, W_neigh, W_self, b_self):
    table = feat.reshape(N_NODES * 2, DH)
    zacc = jnp.zeros((ROWS_PER_TILE, DH), jnp.float32)
    zdeg = jnp.zeros((ROWS_PER_TILE, DEG_W), jnp.float32)
    ones = jnp.ones((CHUNK, DEG_W), jnp.float32)

    acc, deg = _sc_aggregate(edge_index, table, zacc, zdeg, ones)

    return _tc_combine(feat, acc, deg,
                       W_neigh[:, :DH], W_neigh[:, DH:],
                       W_self, b_self.reshape(1, D))
